# baseline (speedup 1.0000x reference)
import functools

import jax
import jax.numpy as jnp
from jax import lax
from jax.experimental import pallas as pl
from jax.experimental.pallas import tpu as pltpu
from jax.experimental.pallas import tpu_sc as plsc


_CHUNK_ROWS = 16
_NBUF = 4


def _make_gather(V, D, B):
    info = plsc.get_sparse_core_info()
    NC, NS = info.num_cores, info.num_subcores
    NW = NC * NS
    assert B % NW == 0
    b_per_w = B // NW
    C = _CHUNK_ROWS
    NBUF = _NBUF
    n_chunks = b_per_w // C
    ni = n_chunks // NBUF
    mesh = plsc.VectorSubcoreMesh(core_axis_name="c", subcore_axis_name="s")

    @functools.partial(
        pl.kernel,
        mesh=mesh,
        out_type=jax.ShapeDtypeStruct((B, D), jnp.float32),
        scratch_types=[
            pltpu.VMEM((n_chunks, C), jnp.int32),
            pltpu.VMEM((NBUF, C, D), jnp.float32),
        ]
        + [pltpu.SemaphoreType.DMA] * (2 * NBUF),
    )
    def k(idx_hbm, table_hbm, out_hbm, idx_v, rows_v, *sems):
        gsem, osem = sems[:NBUF], sems[NBUF:]
        wid = lax.axis_index("s") * NC + lax.axis_index("c")
        base = wid * b_per_w
        pltpu.sync_copy(idx_hbm.at[wid], idx_v)
        bufs = [rows_v.at[b] for b in range(NBUF)]

        def start_gather(g, b, sem):
            pltpu.async_copy(table_hbm.at[idx_v.at[g]], bufs[b], sem)

        def wait_gather(b, sem):
            pltpu.make_async_copy(table_hbm.at[pl.ds(0, C)], bufs[b], sem).wait()

        def start_out(g, b, sem):
            pltpu.async_copy(bufs[b], out_hbm.at[pl.ds(base + g * C, C)], sem)

        def wait_out(b, sem):
            pltpu.make_async_copy(bufs[b], out_hbm.at[pl.ds(0, C)], sem).wait()

        for g in range(NBUF - 1):
            start_gather(g, g, gsem[g])

        def step(i, b, first, last):
            g = NBUF * i + b
            bn = (b + NBUF - 1) % NBUF
            if not (first and b == 0):
                if not (last and b >= 1):
                    wait_out(bn, osem[bn])
            if not (last and b >= 1):
                start_gather(g + NBUF - 1, bn, gsem[bn])
            wait_gather(b, gsem[b])
            start_out(g, b, osem[b])

        for b in range(NBUF):
            step(0, b, first=True, last=ni == 1)

        def body(i, carry):
            for b in range(NBUF):
                step(i, b, first=False, last=False)
            return carry

        if ni > 2:
            lax.fori_loop(1, ni - 1, body, 0)
        if ni > 1:
            for b in range(NBUF):
                step(ni - 1, b, first=False, last=True)
        for b in range(NBUF):
            wait_out(b, osem[b])

    return k


@jax.jit
def kernel(positions, pos_embedding):
    V, D = pos_embedding.shape
    B = positions.size
    info = plsc.get_sparse_core_info()
    NW = info.num_cores * info.num_subcores
    C = _CHUNK_ROWS
    idx = positions.reshape(NW, (B // NW) // C, C).astype(jnp.int32)
    out = _make_gather(V, D, B)(idx, pos_embedding)
    return out.reshape(positions.shape + (D,))


# speedup vs baseline: 1.0054x; 1.0054x over previous
import functools

import jax
import jax.numpy as jnp
from jax import lax
from jax.experimental import pallas as pl
from jax.experimental.pallas import tpu as pltpu
from jax.experimental.pallas import tpu_sc as plsc


_CHUNK_ROWS = 16
_NBUF = 4


def _make_gather(V, D, B):
    info = plsc.get_sparse_core_info()
    NC, NS = info.num_cores, info.num_subcores
    NW = NC * NS
    assert B % NW == 0
    b_per_w = B // NW
    C = _CHUNK_ROWS
    NBUF = _NBUF
    n_chunks = b_per_w // C
    ni = n_chunks // NBUF
    mesh = plsc.VectorSubcoreMesh(core_axis_name="c", subcore_axis_name="s")

    @functools.partial(
        pl.kernel,
        mesh=mesh,
        out_type=jax.ShapeDtypeStruct((B, D), jnp.float32),
        scratch_types=[
            pltpu.VMEM((n_chunks, C), jnp.int32),
            pltpu.VMEM((NBUF, C, D), jnp.float32),
        ]
        + [pltpu.SemaphoreType.DMA] * (2 * NBUF),
    )
    def k(idx_hbm, table_hbm, out_hbm, idx_v, rows_v, *sems):
        gsem, osem = sems[:NBUF], sems[NBUF:]
        wid = lax.axis_index("s") * NC + lax.axis_index("c")
        base = wid * b_per_w
        pltpu.sync_copy(idx_hbm.at[wid], idx_v)
        bufs = [rows_v.at[b] for b in range(NBUF)]

        def start_gather(g, b, sem):
            pltpu.async_copy(table_hbm.at[idx_v.at[g]], bufs[b], sem)

        def wait_gather(b, sem):
            pltpu.make_async_copy(table_hbm.at[pl.ds(0, C)], bufs[b], sem).wait()

        def start_out(g, b, sem):
            pltpu.async_copy(bufs[b], out_hbm.at[pl.ds(base + g * C, C)], sem)

        def wait_out(b, sem):
            pltpu.make_async_copy(bufs[b], out_hbm.at[pl.ds(0, C)], sem).wait()

        for g in range(NBUF - 1):
            start_gather(g, g, gsem[g])

        def body(i, carry):
            for b in range(NBUF):
                g = NBUF * i + b
                bn = (b + NBUF - 1) % NBUF

                @pl.when(jnp.logical_and(g >= 1, g + NBUF - 1 < n_chunks))
                def _():
                    wait_out(bn, osem[bn])

                @pl.when(g + NBUF - 1 < n_chunks)
                def _():
                    start_gather(g + NBUF - 1, bn, gsem[bn])

                wait_gather(b, gsem[b])
                start_out(g, b, osem[b])
            return carry

        lax.fori_loop(0, ni, body, 0)
        for b in range(NBUF):
            wait_out(b, osem[b])

    return k


@jax.jit
def kernel(positions, pos_embedding):
    V, D = pos_embedding.shape
    B = positions.size
    info = plsc.get_sparse_core_info()
    NW = info.num_cores * info.num_subcores
    C = _CHUNK_ROWS
    idx = positions.reshape(NW, (B // NW) // C, C).astype(jnp.int32)
    out = _make_gather(V, D, B)(idx, pos_embedding)
    return out.reshape(positions.shape + (D,))
